# baseline (speedup 1.0000x reference)
import functools

import jax
import jax.numpy as jnp
from jax import lax
from jax.experimental import pallas as pl
from jax.experimental.pallas import tpu as pltpu
from jax.experimental.pallas import tpu_sc as plsc

N = 10000
E = 320000
NODE_DIM = 128
DIM = 32
B = 64

NC = 2
NS = 16
NW = NC * NS
CHUNK = 128
KBUF = 8
RBUF = KBUF * CHUNK
NCH = 80
ROUNDS = NCH // KBUF
PER_W = NCH * CHUNK
EP = NW * PER_W
NROWS = 10240
STRIPE = NROWS // NS
EB = 1024

_f32 = jnp.float32


def _dot(a, b, ca, cb):
    return lax.dot_general(a, b, (((ca,), (cb,)), ((), ())),
                           preferred_element_type=_f32)


def _lin0_body(x_ref, w_ref, b_ref, o_ref):
    o_ref[...] = jax.nn.relu(_dot(x_ref[...], w_ref[...], 1, 1) + b_ref[...])


def _ewbuild_body(ea_ref, w1_ref, b1_ref, w2_ref, b2_ref, ew_ref):
    h1 = jax.nn.relu(_dot(ea_ref[...], w1_ref[...], 1, 1) + b1_ref[...])
    ew_ref[...] = _dot(h1, w2_ref[...], 1, 1) + b2_ref[...]


def _matvec_body(a_ref, ew_ref, m_ref):
    a = a_ref[...]
    ew = ew_ref[...]
    acc = a[:, 0:1] * ew[:, 0:DIM]
    for i in range(1, DIM):
        acc = acc + a[:, i:i + 1] * ew[:, i * DIM:(i + 1) * DIM]
    m_ref[...] = acc


def _update_body(agg2_ref, h_ref, rw_ref, cb_ref, wih_ref, whh_ref,
                 bih_ref, bhh_ref, o_ref):
    agg = agg2_ref[0, :N, :] + agg2_ref[1, :N, :]
    h = h_ref[...]
    m = jax.nn.relu(agg + _dot(h, rw_ref[...], 1, 1) + cb_ref[...])
    gi = _dot(m, wih_ref[...], 1, 1) + bih_ref[...]
    gh = _dot(h, whh_ref[...], 1, 1) + bhh_ref[...]
    r = jax.nn.sigmoid(gi[:, :DIM] + gh[:, :DIM])
    z = jax.nn.sigmoid(gi[:, DIM:2 * DIM] + gh[:, DIM:2 * DIM])
    n = jnp.tanh(gi[:, 2 * DIM:] + r * gh[:, 2 * DIM:])
    o_ref[...] = (1.0 - z) * n + z * h


def _set2set_body(out_ref, bc_ref, wih_ref, whh_ref, bih_ref, bhh_ref,
                  l1w_ref, l1b_ref, l2w_ref, l2b_ref, y_ref):
    out = out_ref[...]
    bc = bc_ref[...]
    seg = lax.broadcasted_iota(jnp.int32, (N, B), 1)
    onehot = jnp.where(bc == seg, 1.0, 0.0).astype(_f32)
    q_star = jnp.zeros((B, 2 * DIM), _f32)
    hx = jnp.zeros((B, DIM), _f32)
    cx = jnp.zeros((B, DIM), _f32)
    for _ in range(3):
        gates = (_dot(q_star, wih_ref[...], 1, 1) + bih_ref[...]
                 + _dot(hx, whh_ref[...], 1, 1) + bhh_ref[...])
        ig = jax.nn.sigmoid(gates[:, :DIM])
        fg = jax.nn.sigmoid(gates[:, DIM:2 * DIM])
        gg = jnp.tanh(gates[:, 2 * DIM:3 * DIM])
        og = jax.nn.sigmoid(gates[:, 3 * DIM:])
        cx = fg * cx + ig * gg
        hx = og * jnp.tanh(cx)
        hxrows = _dot(onehot, hx, 1, 0)
        e = jnp.sum(out * hxrows, axis=1, keepdims=True)
        evals = jnp.where(onehot > 0, e, -1e30)
        emax = jnp.max(evals, axis=0, keepdims=True)
        emax_rows = _dot(onehot, emax, 1, 1)
        ex = jnp.exp(e - emax_rows)
        denom = _dot(onehot, ex, 0, 0)
        denom_rows = _dot(onehot, denom, 1, 0)
        a = ex / (denom_rows + 1e-16)
        r_pool = _dot(onehot, a * out, 0, 0)
        q_star = jnp.concatenate([hx, r_pool], axis=1)
    y = jax.nn.relu(_dot(q_star, l1w_ref[...], 1, 1) + l1b_ref[...])
    y_ref[...] = jnp.sum(y * l2w_ref[...], axis=1, keepdims=True) + l2b_ref[...]


_mesh = plsc.VectorSubcoreMesh(core_axis_name="c", subcore_axis_name="s")


@functools.partial(
    pl.kernel, mesh=_mesh,
    out_type=jax.ShapeDtypeStruct((EP, DIM), _f32),
    scratch_types=[
        pltpu.VMEM((NCH, CHUNK), jnp.int32),
        pltpu.VMEM((RBUF, DIM), _f32),
        pltpu.SemaphoreType.DMA,
    ],
    compiler_params=pltpu.CompilerParams(use_tc_tiling_on_sc=False),
)
def _sc_gather(table_hbm, idx_hbm, out_hbm, idx_v, rows_v, sem):
    wid = lax.axis_index("s") * NC + lax.axis_index("c")
    base = wid * PER_W
    pltpu.sync_copy(idx_hbm.at[wid], idx_v)

    def round_body(g, carry):
        cps = []
        for b in range(KBUF):
            j = g * KBUF + b
            cps.append(pltpu.async_copy(
                table_hbm.at[idx_v.at[j]],
                rows_v.at[pl.ds(b * CHUNK, CHUNK)], sem))
        for cp in cps:
            cp.wait()
        pltpu.sync_copy(rows_v, out_hbm.at[pl.ds(base + g * RBUF, RBUF)])
        return carry

    lax.fori_loop(0, ROUNDS, round_body, 0)


@functools.partial(
    pl.kernel, mesh=_mesh,
    out_type=jax.ShapeDtypeStruct((NC, NROWS, DIM), _f32),
    scratch_types=[
        pltpu.VMEM((NCH, CHUNK), jnp.int32),
        pltpu.VMEM((RBUF, DIM), _f32),
        pltpu.VMEM_SHARED((NROWS, DIM), _f32),
    ],
    compiler_params=pltpu.CompilerParams(use_tc_tiling_on_sc=False),
)
def _sc_scatter(msg_hbm, idx_hbm, zeros_hbm, out_hbm, idx_v, msg_v, acc):
    cid = lax.axis_index("c")
    sid = lax.axis_index("s")
    wid = sid * NC + cid
    base = wid * PER_W
    pltpu.sync_copy(zeros_hbm.at[pl.ds(sid * STRIPE, STRIPE)],
                    acc.at[pl.ds(sid * STRIPE, STRIPE)])
    plsc.subcore_barrier()
    pltpu.sync_copy(idx_hbm.at[wid], idx_v)

    def round_body(g, carry):
        pltpu.sync_copy(msg_hbm.at[pl.ds(base + g * RBUF, RBUF)], msg_v)
        for b in range(KBUF):
            j = g * KBUF + b
            pltpu.sync_copy(msg_v.at[pl.ds(b * CHUNK, CHUNK)],
                            acc.at[idx_v.at[j]], add=True)
        return carry

    lax.fori_loop(0, ROUNDS, round_body, 0)
    plsc.subcore_barrier()
    pltpu.sync_copy(acc.at[pl.ds(sid * STRIPE, STRIPE)],
                    msg_v.at[pl.ds(0, STRIPE)])
    pltpu.sync_copy(msg_v.at[pl.ds(0, STRIPE)],
                    out_hbm.at[cid, pl.ds(sid * STRIPE, STRIPE)])


def kernel(x, edge_index, edge_attr, batch, lin0_W, lin0_b, nn1_W, nn1_b,
           nn2_W, nn2_b, root_W, conv_b, gru_W_ih, gru_W_hh, gru_b_ih,
           gru_b_hh, lstm_W_ih, lstm_W_hh, lstm_b_ih, lstm_b_hh, lin1_W,
           lin1_b, lin2_W, lin2_b):
    src = edge_index[0].astype(jnp.int32)
    dst = edge_index[1].astype(jnp.int32)
    pad = EP - E
    src_p = jnp.concatenate([src, jnp.zeros((pad,), jnp.int32)])
    src_p = src_p.reshape(NW, NCH, CHUNK)
    dst_p = jnp.concatenate([dst, jnp.full((pad,), N, jnp.int32)])
    dst_p = dst_p.reshape(NW, NCH, CHUNK)
    ea_p = jnp.pad(edge_attr, ((0, pad), (0, 2)))
    nn1_Wp = jnp.pad(nn1_W, ((0, 0), (0, 2)))
    zeros_rows = jnp.zeros((NROWS, DIM), _f32)

    h = pl.pallas_call(
        _lin0_body,
        out_shape=jax.ShapeDtypeStruct((N, DIM), _f32),
    )(x, lin0_W, lin0_b.reshape(1, DIM))

    grid_e = EP // EB
    ew = pl.pallas_call(
        _ewbuild_body,
        grid=(grid_e,),
        in_specs=[
            pl.BlockSpec((EB, 8), lambda i: (i, 0)),
            pl.BlockSpec((128, 8), lambda i: (0, 0)),
            pl.BlockSpec((1, 128), lambda i: (0, 0)),
            pl.BlockSpec((DIM * DIM, 128), lambda i: (0, 0)),
            pl.BlockSpec((1, DIM * DIM), lambda i: (0, 0)),
        ],
        out_specs=pl.BlockSpec((EB, DIM * DIM), lambda i: (i, 0)),
        out_shape=jax.ShapeDtypeStruct((EP, DIM * DIM), _f32),
    )(ea_p, nn1_Wp, nn1_b.reshape(1, 128), nn2_W,
      nn2_b.reshape(1, DIM * DIM))

    matvec = pl.pallas_call(
        _matvec_body,
        grid=(grid_e,),
        in_specs=[
            pl.BlockSpec((EB, DIM), lambda i: (i, 0)),
            pl.BlockSpec((EB, DIM * DIM), lambda i: (i, 0)),
        ],
        out_specs=pl.BlockSpec((EB, DIM), lambda i: (i, 0)),
        out_shape=jax.ShapeDtypeStruct((EP, DIM), _f32),
    )

    update = pl.pallas_call(
        _update_body,
        out_shape=jax.ShapeDtypeStruct((N, DIM), _f32),
    )

    for _ in range(5):
        out_src = _sc_gather(h, src_p)
        msg = matvec(out_src, ew)
        agg2 = _sc_scatter(msg, dst_p, zeros_rows)
        h = update(agg2, h, root_W, conv_b.reshape(1, DIM), gru_W_ih,
                   gru_W_hh, gru_b_ih.reshape(1, 3 * DIM),
                   gru_b_hh.reshape(1, 3 * DIM))

    y = pl.pallas_call(
        _set2set_body,
        out_shape=jax.ShapeDtypeStruct((B, 1), _f32),
    )(h, batch.astype(jnp.int32).reshape(N, 1), lstm_W_ih, lstm_W_hh,
      lstm_b_ih.reshape(1, 4 * DIM), lstm_b_hh.reshape(1, 4 * DIM),
      lin1_W, lin1_b.reshape(1, DIM), lin2_W, lin2_b.reshape(1, 1))
    return y.reshape(-1)


# speedup vs baseline: 2.6898x; 2.6898x over previous
import functools

import jax
import jax.numpy as jnp
from jax import lax
from jax.experimental import pallas as pl
from jax.experimental.pallas import tpu as pltpu
from jax.experimental.pallas import tpu_sc as plsc

N = 10000
E = 320000
NODE_DIM = 128
DIM = 32
B = 64

NC = 2
NS = 16
NW = NC * NS
CHUNK = 128
KBUF = 8
RBUF = KBUF * CHUNK
NCH = 80
ROUNDS = NCH // KBUF
PER_W = NCH * CHUNK
EP = NW * PER_W
NROWS = 10240
STRIPE = NROWS // NS
EB = 1024

_f32 = jnp.float32


def _dot(a, b, ca, cb):
    return lax.dot_general(a, b, (((ca,), (cb,)), ((), ())),
                           preferred_element_type=_f32)


def _lin0_body(x_ref, w_ref, b_ref, o_ref):
    o_ref[...] = jax.nn.relu(_dot(x_ref[...], w_ref[...], 1, 1) + b_ref[...])


def _ewbuild_body(ea_ref, w1_ref, b1_ref, w2_ref, b2_ref, ew_ref):
    h1 = jax.nn.relu(_dot(ea_ref[...], w1_ref[...], 1, 1) + b1_ref[...])
    ew_ref[...] = _dot(h1, w2_ref[...], 1, 1) + b2_ref[...]


def _matvec_body(a_ref, ew_ref, r_ref, s_ref, m_ref):
    a_rep = _dot(a_ref[...], r_ref[...], 1, 0)
    m_ref[...] = _dot(a_rep * ew_ref[...], s_ref[...], 1, 0)


def _update_body(agg2_ref, h_ref, rw_ref, cb_ref, wih_ref, whh_ref,
                 bih_ref, bhh_ref, o_ref):
    agg = agg2_ref[0, :N, :] + agg2_ref[1, :N, :]
    h = h_ref[...]
    m = jax.nn.relu(agg + _dot(h, rw_ref[...], 1, 1) + cb_ref[...])
    gi = _dot(m, wih_ref[...], 1, 1) + bih_ref[...]
    gh = _dot(h, whh_ref[...], 1, 1) + bhh_ref[...]
    r = jax.nn.sigmoid(gi[:, :DIM] + gh[:, :DIM])
    z = jax.nn.sigmoid(gi[:, DIM:2 * DIM] + gh[:, DIM:2 * DIM])
    n = jnp.tanh(gi[:, 2 * DIM:] + r * gh[:, 2 * DIM:])
    o_ref[...] = (1.0 - z) * n + z * h


def _set2set_body(out_ref, bc_ref, wih_ref, whh_ref, bih_ref, bhh_ref,
                  l1w_ref, l1b_ref, l2w_ref, l2b_ref, y_ref):
    out = out_ref[...]
    bc = bc_ref[...]
    seg = lax.broadcasted_iota(jnp.int32, (N, B), 1)
    onehot = jnp.where(bc == seg, 1.0, 0.0).astype(_f32)
    q_star = jnp.zeros((B, 2 * DIM), _f32)
    hx = jnp.zeros((B, DIM), _f32)
    cx = jnp.zeros((B, DIM), _f32)
    for _ in range(3):
        gates = (_dot(q_star, wih_ref[...], 1, 1) + bih_ref[...]
                 + _dot(hx, whh_ref[...], 1, 1) + bhh_ref[...])
        ig = jax.nn.sigmoid(gates[:, :DIM])
        fg = jax.nn.sigmoid(gates[:, DIM:2 * DIM])
        gg = jnp.tanh(gates[:, 2 * DIM:3 * DIM])
        og = jax.nn.sigmoid(gates[:, 3 * DIM:])
        cx = fg * cx + ig * gg
        hx = og * jnp.tanh(cx)
        hxrows = _dot(onehot, hx, 1, 0)
        e = jnp.sum(out * hxrows, axis=1, keepdims=True)
        evals = jnp.where(onehot > 0, e, -1e30)
        emax = jnp.max(evals, axis=0, keepdims=True)
        emax_rows = _dot(onehot, emax, 1, 1)
        ex = jnp.exp(e - emax_rows)
        denom = _dot(onehot, ex, 0, 0)
        denom_rows = _dot(onehot, denom, 1, 0)
        a = ex / (denom_rows + 1e-16)
        r_pool = _dot(onehot, a * out, 0, 0)
        q_star = jnp.concatenate([hx, r_pool], axis=1)
    y = jax.nn.relu(_dot(q_star, l1w_ref[...], 1, 1) + l1b_ref[...])
    y_ref[...] = jnp.sum(y * l2w_ref[...], axis=1, keepdims=True) + l2b_ref[...]


_mesh = plsc.VectorSubcoreMesh(core_axis_name="c", subcore_axis_name="s")


@functools.partial(
    pl.kernel, mesh=_mesh,
    out_type=jax.ShapeDtypeStruct((EP, DIM), _f32),
    scratch_types=[
        pltpu.VMEM((NCH, CHUNK), jnp.int32),
        pltpu.VMEM((RBUF, DIM), _f32),
        pltpu.SemaphoreType.DMA,
    ],
    compiler_params=pltpu.CompilerParams(use_tc_tiling_on_sc=False),
)
def _sc_gather(table_hbm, idx_hbm, out_hbm, idx_v, rows_v, sem):
    wid = lax.axis_index("s") * NC + lax.axis_index("c")
    base = wid * PER_W
    pltpu.sync_copy(idx_hbm.at[wid], idx_v)

    def round_body(g, carry):
        cps = []
        for b in range(KBUF):
            j = g * KBUF + b
            cps.append(pltpu.async_copy(
                table_hbm.at[idx_v.at[j]],
                rows_v.at[pl.ds(b * CHUNK, CHUNK)], sem))
        for cp in cps:
            cp.wait()
        pltpu.sync_copy(rows_v, out_hbm.at[pl.ds(base + g * RBUF, RBUF)])
        return carry

    lax.fori_loop(0, ROUNDS, round_body, 0)


@functools.partial(
    pl.kernel, mesh=_mesh,
    out_type=jax.ShapeDtypeStruct((NC, NROWS, DIM), _f32),
    scratch_types=[
        pltpu.VMEM((NCH, CHUNK), jnp.int32),
        pltpu.VMEM((RBUF, DIM), _f32),
        pltpu.VMEM_SHARED((NROWS, DIM), _f32),
    ],
    compiler_params=pltpu.CompilerParams(use_tc_tiling_on_sc=False),
)
def _sc_scatter(msg_hbm, idx_hbm, zeros_hbm, out_hbm, idx_v, msg_v, acc):
    cid = lax.axis_index("c")
    sid = lax.axis_index("s")
    wid = sid * NC + cid
    base = wid * PER_W
    pltpu.sync_copy(zeros_hbm.at[pl.ds(sid * STRIPE, STRIPE)],
                    acc.at[pl.ds(sid * STRIPE, STRIPE)])
    plsc.subcore_barrier()
    pltpu.sync_copy(idx_hbm.at[wid], idx_v)

    def round_body(g, carry):
        pltpu.sync_copy(msg_hbm.at[pl.ds(base + g * RBUF, RBUF)], msg_v)
        for b in range(KBUF):
            j = g * KBUF + b
            pltpu.sync_copy(msg_v.at[pl.ds(b * CHUNK, CHUNK)],
                            acc.at[idx_v.at[j]], add=True)
        return carry

    lax.fori_loop(0, ROUNDS, round_body, 0)
    plsc.subcore_barrier()
    pltpu.sync_copy(acc.at[pl.ds(sid * STRIPE, STRIPE)],
                    msg_v.at[pl.ds(0, STRIPE)])
    pltpu.sync_copy(msg_v.at[pl.ds(0, STRIPE)],
                    out_hbm.at[cid, pl.ds(sid * STRIPE, STRIPE)])


def kernel(x, edge_index, edge_attr, batch, lin0_W, lin0_b, nn1_W, nn1_b,
           nn2_W, nn2_b, root_W, conv_b, gru_W_ih, gru_W_hh, gru_b_ih,
           gru_b_hh, lstm_W_ih, lstm_W_hh, lstm_b_ih, lstm_b_hh, lin1_W,
           lin1_b, lin2_W, lin2_b):
    src = edge_index[0].astype(jnp.int32)
    dst = edge_index[1].astype(jnp.int32)
    pad = EP - E
    src_p = jnp.concatenate([src, jnp.zeros((pad,), jnp.int32)])
    src_p = src_p.reshape(NW, NCH, CHUNK)
    dst_p = jnp.concatenate([dst, jnp.full((pad,), N, jnp.int32)])
    dst_p = dst_p.reshape(NW, NCH, CHUNK)
    ea_p = jnp.pad(edge_attr, ((0, pad), (0, 2)))
    nn1_Wp = jnp.pad(nn1_W, ((0, 0), (0, 2)))
    zeros_rows = jnp.zeros((NROWS, DIM), _f32)

    h = pl.pallas_call(
        _lin0_body,
        out_shape=jax.ShapeDtypeStruct((N, DIM), _f32),
    )(x, lin0_W, lin0_b.reshape(1, DIM))

    grid_e = EP // EB
    ew = pl.pallas_call(
        _ewbuild_body,
        grid=(grid_e,),
        in_specs=[
            pl.BlockSpec((EB, 8), lambda i: (i, 0)),
            pl.BlockSpec((128, 8), lambda i: (0, 0)),
            pl.BlockSpec((1, 128), lambda i: (0, 0)),
            pl.BlockSpec((DIM * DIM, 128), lambda i: (0, 0)),
            pl.BlockSpec((1, DIM * DIM), lambda i: (0, 0)),
        ],
        out_specs=pl.BlockSpec((EB, DIM * DIM), lambda i: (i, 0)),
        out_shape=jax.ShapeDtypeStruct((EP, DIM * DIM), _f32),
    )(ea_p, nn1_Wp, nn1_b.reshape(1, 128), nn2_W,
      nn2_b.reshape(1, DIM * DIM))

    ii = jnp.arange(DIM * DIM, dtype=jnp.int32)
    Rmat = (ii[None, :] // DIM == jnp.arange(DIM, dtype=jnp.int32)[:, None]
            ).astype(_f32)
    Smat = (ii[:, None] % DIM == jnp.arange(DIM, dtype=jnp.int32)[None, :]
            ).astype(_f32)

    matvec = pl.pallas_call(
        _matvec_body,
        grid=(grid_e,),
        in_specs=[
            pl.BlockSpec((EB, DIM), lambda i: (i, 0)),
            pl.BlockSpec((EB, DIM * DIM), lambda i: (i, 0)),
            pl.BlockSpec((DIM, DIM * DIM), lambda i: (0, 0)),
            pl.BlockSpec((DIM * DIM, DIM), lambda i: (0, 0)),
        ],
        out_specs=pl.BlockSpec((EB, DIM), lambda i: (i, 0)),
        out_shape=jax.ShapeDtypeStruct((EP, DIM), _f32),
    )

    update = pl.pallas_call(
        _update_body,
        out_shape=jax.ShapeDtypeStruct((N, DIM), _f32),
    )

    for _ in range(5):
        out_src = _sc_gather(h, src_p)
        msg = matvec(out_src, ew, Rmat, Smat)
        agg2 = _sc_scatter(msg, dst_p, zeros_rows)
        h = update(agg2, h, root_W, conv_b.reshape(1, DIM), gru_W_ih,
                   gru_W_hh, gru_b_ih.reshape(1, 3 * DIM),
                   gru_b_hh.reshape(1, 3 * DIM))

    y = pl.pallas_call(
        _set2set_body,
        out_shape=jax.ShapeDtypeStruct((B, 1), _f32),
    )(h, batch.astype(jnp.int32).reshape(N, 1), lstm_W_ih, lstm_W_hh,
      lstm_b_ih.reshape(1, 4 * DIM), lstm_b_hh.reshape(1, 4 * DIM),
      lin1_W, lin1_b.reshape(1, DIM), lin2_W, lin2_b.reshape(1, 1))
    return y.reshape(-1)


# speedup vs baseline: 3.2056x; 1.1917x over previous
import functools

import jax
import jax.numpy as jnp
from jax import lax
from jax.experimental import pallas as pl
from jax.experimental.pallas import tpu as pltpu
from jax.experimental.pallas import tpu_sc as plsc

N = 10000
E = 320000
NODE_DIM = 128
DIM = 32
B = 64

NC = 2
NS = 16
NW = NC * NS
CHUNK = 128
KBUF = 8
RBUF = KBUF * CHUNK
NCH = 80
ROUNDS = NCH // KBUF
PER_W = NCH * CHUNK
EP = NW * PER_W
NROWS = 10240
STRIPE = NROWS // NS
EB = 1024

_f32 = jnp.float32


def _dot(a, b, ca, cb):
    return lax.dot_general(a, b, (((ca,), (cb,)), ((), ())),
                           preferred_element_type=_f32)


def _lin0_body(x_ref, w_ref, b_ref, o_ref):
    o_ref[...] = jax.nn.relu(_dot(x_ref[...], w_ref[...], 1, 1) + b_ref[...])


def _ewbuild_matvec_body(ea_ref, w1_ref, b1_ref, w2_ref, b2_ref, a_ref,
                         r_ref, s_ref, ew_ref, m_ref):
    h1 = jax.nn.relu(_dot(ea_ref[...], w1_ref[...], 1, 1) + b1_ref[...])
    ewf = _dot(h1, w2_ref[...], 1, 1) + b2_ref[...]
    ew_ref[...] = ewf.astype(jnp.bfloat16)
    a_rep = _dot(a_ref[...], r_ref[...], 1, 0)
    m_ref[...] = _dot(a_rep * ewf, s_ref[...], 1, 0)


def _matvec_body(a_ref, ew_ref, r_ref, s_ref, m_ref):
    a_rep = _dot(a_ref[...], r_ref[...], 1, 0)
    m_ref[...] = _dot(a_rep * ew_ref[...].astype(_f32), s_ref[...], 1, 0)


def _update_body(agg2_ref, h_ref, rw_ref, cb_ref, wih_ref, whh_ref,
                 bih_ref, bhh_ref, o_ref):
    agg = agg2_ref[0, :N, :] + agg2_ref[1, :N, :]
    h = h_ref[...]
    m = jax.nn.relu(agg + _dot(h, rw_ref[...], 1, 1) + cb_ref[...])
    gi = _dot(m, wih_ref[...], 1, 1) + bih_ref[...]
    gh = _dot(h, whh_ref[...], 1, 1) + bhh_ref[...]
    r = jax.nn.sigmoid(gi[:, :DIM] + gh[:, :DIM])
    z = jax.nn.sigmoid(gi[:, DIM:2 * DIM] + gh[:, DIM:2 * DIM])
    n = jnp.tanh(gi[:, 2 * DIM:] + r * gh[:, 2 * DIM:])
    o_ref[...] = (1.0 - z) * n + z * h


def _set2set_body(out_ref, bc_ref, wih_ref, whh_ref, bih_ref, bhh_ref,
                  l1w_ref, l1b_ref, l2w_ref, l2b_ref, y_ref):
    out = out_ref[...]
    bc = bc_ref[...]
    seg = lax.broadcasted_iota(jnp.int32, (N, B), 1)
    onehot = jnp.where(bc == seg, 1.0, 0.0).astype(_f32)
    q_star = jnp.zeros((B, 2 * DIM), _f32)
    hx = jnp.zeros((B, DIM), _f32)
    cx = jnp.zeros((B, DIM), _f32)
    for _ in range(3):
        gates = (_dot(q_star, wih_ref[...], 1, 1) + bih_ref[...]
                 + _dot(hx, whh_ref[...], 1, 1) + bhh_ref[...])
        ig = jax.nn.sigmoid(gates[:, :DIM])
        fg = jax.nn.sigmoid(gates[:, DIM:2 * DIM])
        gg = jnp.tanh(gates[:, 2 * DIM:3 * DIM])
        og = jax.nn.sigmoid(gates[:, 3 * DIM:])
        cx = fg * cx + ig * gg
        hx = og * jnp.tanh(cx)
        hxrows = _dot(onehot, hx, 1, 0)
        e = jnp.sum(out * hxrows, axis=1, keepdims=True)
        evals = jnp.where(onehot > 0, e, -1e30)
        emax = jnp.max(evals, axis=0, keepdims=True)
        emax_rows = _dot(onehot, emax, 1, 1)
        ex = jnp.exp(e - emax_rows)
        denom = _dot(onehot, ex, 0, 0)
        denom_rows = _dot(onehot, denom, 1, 0)
        a = ex / (denom_rows + 1e-16)
        r_pool = _dot(onehot, a * out, 0, 0)
        q_star = jnp.concatenate([hx, r_pool], axis=1)
    y = jax.nn.relu(_dot(q_star, l1w_ref[...], 1, 1) + l1b_ref[...])
    y_ref[...] = jnp.sum(y * l2w_ref[...], axis=1, keepdims=True) + l2b_ref[...]


_mesh = plsc.VectorSubcoreMesh(core_axis_name="c", subcore_axis_name="s")


@functools.partial(
    pl.kernel, mesh=_mesh,
    out_type=jax.ShapeDtypeStruct((EP, DIM), _f32),
    scratch_types=[
        pltpu.VMEM((NCH, CHUNK), jnp.int32),
        pltpu.VMEM((RBUF, DIM), _f32),
        pltpu.SemaphoreType.DMA,
    ],
    compiler_params=pltpu.CompilerParams(use_tc_tiling_on_sc=False),
)
def _sc_gather(table_hbm, idx_hbm, out_hbm, idx_v, rows_v, sem):
    wid = lax.axis_index("s") * NC + lax.axis_index("c")
    base = wid * PER_W
    pltpu.sync_copy(idx_hbm.at[wid], idx_v)

    def round_body(g, carry):
        cps = []
        for b in range(KBUF):
            j = g * KBUF + b
            cps.append(pltpu.async_copy(
                table_hbm.at[idx_v.at[j]],
                rows_v.at[pl.ds(b * CHUNK, CHUNK)], sem))
        for cp in cps:
            cp.wait()
        pltpu.sync_copy(rows_v, out_hbm.at[pl.ds(base + g * RBUF, RBUF)])
        return carry

    lax.fori_loop(0, ROUNDS, round_body, 0)


@functools.partial(
    pl.kernel, mesh=_mesh,
    out_type=jax.ShapeDtypeStruct((NC, NROWS, DIM), _f32),
    scratch_types=[
        pltpu.VMEM((NCH, CHUNK), jnp.int32),
        pltpu.VMEM((RBUF, DIM), _f32),
        pltpu.VMEM_SHARED((NROWS, DIM), _f32),
    ],
    compiler_params=pltpu.CompilerParams(use_tc_tiling_on_sc=False),
)
def _sc_scatter(msg_hbm, idx_hbm, zeros_hbm, out_hbm, idx_v, msg_v, acc):
    cid = lax.axis_index("c")
    sid = lax.axis_index("s")
    wid = sid * NC + cid
    base = wid * PER_W
    pltpu.sync_copy(zeros_hbm.at[pl.ds(sid * STRIPE, STRIPE)],
                    acc.at[pl.ds(sid * STRIPE, STRIPE)])
    plsc.subcore_barrier()
    pltpu.sync_copy(idx_hbm.at[wid], idx_v)

    def round_body(g, carry):
        pltpu.sync_copy(msg_hbm.at[pl.ds(base + g * RBUF, RBUF)], msg_v)
        for b in range(KBUF):
            j = g * KBUF + b
            pltpu.sync_copy(msg_v.at[pl.ds(b * CHUNK, CHUNK)],
                            acc.at[idx_v.at[j]], add=True)
        return carry

    lax.fori_loop(0, ROUNDS, round_body, 0)
    plsc.subcore_barrier()
    pltpu.sync_copy(acc.at[pl.ds(sid * STRIPE, STRIPE)],
                    msg_v.at[pl.ds(0, STRIPE)])
    pltpu.sync_copy(msg_v.at[pl.ds(0, STRIPE)],
                    out_hbm.at[cid, pl.ds(sid * STRIPE, STRIPE)])


def kernel(x, edge_index, edge_attr, batch, lin0_W, lin0_b, nn1_W, nn1_b,
           nn2_W, nn2_b, root_W, conv_b, gru_W_ih, gru_W_hh, gru_b_ih,
           gru_b_hh, lstm_W_ih, lstm_W_hh, lstm_b_ih, lstm_b_hh, lin1_W,
           lin1_b, lin2_W, lin2_b):
    src = edge_index[0].astype(jnp.int32)
    dst = edge_index[1].astype(jnp.int32)
    pad = EP - E
    src_p = jnp.concatenate([src, jnp.zeros((pad,), jnp.int32)])
    src_p = src_p.reshape(NW, NCH, CHUNK)
    dst_p = jnp.concatenate([dst, jnp.full((pad,), N, jnp.int32)])
    dst_p = dst_p.reshape(NW, NCH, CHUNK)
    zeros_rows = jnp.zeros((NROWS, DIM), _f32)
    ea_p = jnp.pad(edge_attr, ((0, pad), (0, 0)))

    h = pl.pallas_call(
        _lin0_body,
        out_shape=jax.ShapeDtypeStruct((N, DIM), _f32),
    )(x, lin0_W, lin0_b.reshape(1, DIM))

    grid_e = EP // EB
    ii = jnp.arange(DIM * DIM, dtype=jnp.int32)
    Rmat = (ii[None, :] // DIM == jnp.arange(DIM, dtype=jnp.int32)[:, None]
            ).astype(_f32)
    Smat = (ii[:, None] % DIM == jnp.arange(DIM, dtype=jnp.int32)[None, :]
            ).astype(_f32)

    ewbuild_matvec = pl.pallas_call(
        _ewbuild_matvec_body,
        grid=(grid_e,),
        in_specs=[
            pl.BlockSpec((EB, 6), lambda i: (i, 0)),
            pl.BlockSpec((128, 6), lambda i: (0, 0)),
            pl.BlockSpec((1, 128), lambda i: (0, 0)),
            pl.BlockSpec((DIM * DIM, 128), lambda i: (0, 0)),
            pl.BlockSpec((1, DIM * DIM), lambda i: (0, 0)),
            pl.BlockSpec((EB, DIM), lambda i: (i, 0)),
            pl.BlockSpec((DIM, DIM * DIM), lambda i: (0, 0)),
            pl.BlockSpec((DIM * DIM, DIM), lambda i: (0, 0)),
        ],
        out_specs=[
            pl.BlockSpec((EB, DIM * DIM), lambda i: (i, 0)),
            pl.BlockSpec((EB, DIM), lambda i: (i, 0)),
        ],
        out_shape=[
            jax.ShapeDtypeStruct((EP, DIM * DIM), jnp.bfloat16),
            jax.ShapeDtypeStruct((EP, DIM), _f32),
        ],
    )

    matvec = pl.pallas_call(
        _matvec_body,
        grid=(grid_e,),
        in_specs=[
            pl.BlockSpec((EB, DIM), lambda i: (i, 0)),
            pl.BlockSpec((EB, DIM * DIM), lambda i: (i, 0)),
            pl.BlockSpec((DIM, DIM * DIM), lambda i: (0, 0)),
            pl.BlockSpec((DIM * DIM, DIM), lambda i: (0, 0)),
        ],
        out_specs=pl.BlockSpec((EB, DIM), lambda i: (i, 0)),
        out_shape=jax.ShapeDtypeStruct((EP, DIM), _f32),
    )

    update = pl.pallas_call(
        _update_body,
        out_shape=jax.ShapeDtypeStruct((N, DIM), _f32),
    )

    ew = None
    for it in range(5):
        out_src = _sc_gather(h, src_p)
        if it == 0:
            ew, msg = ewbuild_matvec(ea_p, nn1_W, nn1_b.reshape(1, 128),
                                     nn2_W, nn2_b.reshape(1, DIM * DIM),
                                     out_src, Rmat, Smat)
        else:
            msg = matvec(out_src, ew, Rmat, Smat)
        agg2 = _sc_scatter(msg, dst_p, zeros_rows)
        h = update(agg2, h, root_W, conv_b.reshape(1, DIM), gru_W_ih,
                   gru_W_hh, gru_b_ih.reshape(1, 3 * DIM),
                   gru_b_hh.reshape(1, 3 * DIM))

    y = pl.pallas_call(
        _set2set_body,
        out_shape=jax.ShapeDtypeStruct((B, 1), _f32),
    )(h, batch.astype(jnp.int32).reshape(N, 1), lstm_W_ih, lstm_W_hh,
      lstm_b_ih.reshape(1, 4 * DIM), lstm_b_hh.reshape(1, 4 * DIM),
      lin1_W, lin1_b.reshape(1, DIM), lin2_W, lin2_b.reshape(1, 1))
    return y.reshape(-1)
